# baseline (speedup 1.0000x reference)
import functools

import jax
import jax.numpy as jnp
from jax.experimental import pallas as pl


def _chamfer_body(x1_ref, x2_ref, d1_ref, d2_ref):
    j = pl.program_id(1)

    x1 = x1_ref[0]
    x2 = x2_ref[0]

    sq1 = jnp.sum(x1 * x1, axis=1)
    sq2 = jnp.sum(x2 * x2, axis=1)

    inner = jax.lax.dot_general(
        x1, x2,
        dimension_numbers=(((1,), (1,)), ((), ())),
        preferred_element_type=jnp.float32,
    )

    pd = (sq1[:, None] + sq2[None, :]) - 2.0 * inner

    rowmin = jnp.min(pd, axis=1)
    d2_ref[0, 0] = jnp.min(pd, axis=0)

    @pl.when(j == 0)
    def _():
        d1_ref[0, 0] = rowmin

    @pl.when(j != 0)
    def _():
        d1_ref[0, 0] = jnp.minimum(d1_ref[0, 0], rowmin)


@functools.partial(jax.jit, static_argnames=("interpret",))
def _chamfer(xyz1, xyz2, interpret=False):
    B, N, _ = xyz1.shape
    M = xyz2.shape[1]
    TM = 512

    grid = (B, M // TM)
    return pl.pallas_call(
        _chamfer_body,
        grid=grid,
        in_specs=[
            pl.BlockSpec((1, N, 3), lambda b, j: (b, 0, 0)),
            pl.BlockSpec((1, TM, 3), lambda b, j: (b, j, 0)),
        ],
        out_specs=[
            pl.BlockSpec((1, 1, N), lambda b, j: (b, 0, 0)),
            pl.BlockSpec((1, 1, TM), lambda b, j: (b, 0, j)),
        ],
        out_shape=[
            jax.ShapeDtypeStruct((B, 1, N), jnp.float32),
            jax.ShapeDtypeStruct((B, 1, M), jnp.float32),
        ],
        interpret=interpret,
    )(xyz1, xyz2)


def kernel(xyz1, xyz2):
    if xyz1.ndim == 2:
        xyz1 = xyz1[None]
    if xyz2.ndim == 2:
        xyz2 = xyz2[None]
    d1, d2 = _chamfer(xyz1, xyz2)
    return (d1[:, 0, :], d2[:, 0, :])
